# Initial kernel scaffold; baseline (speedup 1.0000x reference)
import jax
import jax.numpy as jnp
from jax.experimental import pallas as pl


def kernel(x, table):
    raise NotImplementedError("write your pallas kernel here")


# speedup vs baseline: 5.1054x; 5.1054x over previous
import functools

import jax
import jax.numpy as jnp
from jax import lax
from jax.experimental import pallas as pl
from jax.experimental.pallas import tpu as pltpu
from jax.experimental.pallas import tpu_sc as plsc

VOCAB = 1000
EMB = 32
BATCH = 16384
SEQ = 200

B = BATCH * SEQ
NC, NS = 2, 16
NW = NC * NS
IDX_MINOR = 128
CHUNK_DMAS = 8
CHUNK = IDX_MINOR * CHUNK_DMAS
ROWS_PER_W = B // NW
ITERS = ROWS_PER_W // CHUNK
X2D_ROWS_PER_W = ROWS_PER_W // IDX_MINOR


def _emb_kernel(x2d_hbm, table_hbm, out_hbm, idx_v, rows_v, sem):
    wid = lax.axis_index("s") * NC + lax.axis_index("c")
    x2d_base = wid * X2D_ROWS_PER_W
    out_base = wid * ROWS_PER_W

    def body(g, carry):
        pltpu.sync_copy(x2d_hbm.at[pl.ds(x2d_base + g * CHUNK_DMAS, CHUNK_DMAS)],
                        idx_v)
        copies = []
        for j in range(CHUNK_DMAS):
            copies.append(
                pltpu.async_copy(table_hbm.at[idx_v.at[j]],
                                 rows_v.at[pl.ds(j * IDX_MINOR, IDX_MINOR)],
                                 sem))
        for c in copies:
            c.wait()
        pltpu.sync_copy(rows_v, out_hbm.at[pl.ds(out_base + g * CHUNK, CHUNK)])
        return carry

    lax.fori_loop(0, ITERS, body, 0)


@jax.jit
def _run(x2d, table):
    mesh = plsc.VectorSubcoreMesh(core_axis_name="c", subcore_axis_name="s")
    return pl.kernel(
        _emb_kernel,
        mesh=mesh,
        out_type=jax.ShapeDtypeStruct((B, EMB), jnp.float32),
        scratch_types=[
            pltpu.VMEM((CHUNK_DMAS, IDX_MINOR), jnp.int32),
            pltpu.VMEM((CHUNK, EMB), jnp.float32),
            pltpu.SemaphoreType.DMA,
        ],
        compiler_params=pltpu.CompilerParams(use_tc_tiling_on_sc=False),
    )(x2d, table)


def kernel(x, table):
    x2d = x.reshape(B // IDX_MINOR, IDX_MINOR).astype(jnp.int32)
    out = _run(x2d, table)
    return out.reshape(BATCH, SEQ, EMB)
